# baseline (speedup 1.0000x reference)
import functools

import jax
import jax.numpy as jnp
from jax import lax
from jax.experimental import pallas as pl
from jax.experimental.pallas import tpu as pltpu
from jax.experimental.pallas import tpu_sc as plsc

D_MODEL = 1024
POOL = 64
TOPK = 8
D_FF = 2048
MOD = 64
HEADS = 16
DH = D_MODEL // HEADS
S = 2048

BS = 256
QB = 256
NEG = -1e30

NCORES = 2
NSUB = 16
NWORK = NCORES * NSUB
TPW = S // NWORK
LANES = 16


def _dot(a, b):
    return jax.lax.dot_general(a, b, (((1,), (0,)), ((), ())),
                               preferred_element_type=jnp.float32)


def _dot_t(a, b):
    return jax.lax.dot_general(a, b, (((1,), (1,)), ((), ())),
                               preferred_element_type=jnp.float32)


def _bf(t):
    return t.astype(jnp.bfloat16)


def _logits_body(x_ref, wr_ref, br_ref, wc_ref, bc_ref, me_ref, cm_ref):
    x = x_ref[...]
    scores = _dot(x, wr_ref[...]) + br_ref[...]
    ctx = _dot(x, wc_ref[...]) + bc_ref[...]
    modl = _dot_t(ctx, me_ref[...])
    cm_ref[...] = jnp.concatenate([scores.T, modl.T], axis=0)


def _route_sc_body(cm_hbm, w_hbm, rz_hbm, cm_v, out_v, rz_v):
    wid = lax.axis_index("s") * NCORES + lax.axis_index("c")
    slab = (wid // 2) * (2 * TPW)
    toff = (wid % 2) * TPW
    pltpu.sync_copy(cm_hbm.at[:, pl.ds(slab, 2 * TPW)], cm_v)

    for g in range(TPW // LANES):
        tok = pl.ds(toff + g * LANES, LANES)
        otok = pl.ds(g * LANES, LANES)
        neg = jnp.full((LANES,), NEG, jnp.float32)
        one = jnp.full((LANES,), 1, jnp.int32)
        zero = jnp.full((LANES,), 0, jnp.int32)

        def top_step(p, ms):
            ins = cm_v[p, tok]
            out = []
            for j in range(TOPK):
                hi = jnp.maximum(ms[j], ins)
                ins = jnp.minimum(ms[j], ins)
                out.append(hi)
            return tuple(out)

        ms = lax.fori_loop(0, POOL, top_step, tuple([neg] * TOPK),
                           unroll=4)
        m0 = ms[0]
        t8 = ms[TOPK - 1]
        ngt = jnp.zeros((LANES,), jnp.int32)
        for j in range(TOPK):
            ngt = ngt + jnp.where(ms[j] > t8, one, zero)
        quota = TOPK - ngt

        def sel_step(p, carry):
            z, c = carry
            v = cm_v[p, tok]
            eq = v == t8
            inc = jnp.logical_or(v > t8, jnp.logical_and(eq, c < quota))
            e = jnp.where(inc, jnp.exp(v - m0),
                          jnp.zeros((LANES,), jnp.float32))
            ml = cm_v[p + POOL, tok]
            sg = 1.0 / (1.0 + jnp.exp(-ml))
            out_v[p, otok] = e * sg
            return z + e, c + jnp.where(eq, one, zero)

        z, _ = lax.fori_loop(0, POOL, sel_step,
                             (jnp.zeros((LANES,), jnp.float32),
                              jnp.zeros((LANES,), jnp.int32)),
                             unroll=4)
        rz_v[otok] = 1.0 / z

    pltpu.sync_copy(out_v, w_hbm.at[wid])
    pltpu.sync_copy(rz_v, rz_hbm.at[wid])


_route_sc_cache = []


def _get_route_sc():
    if not _route_sc_cache:
        _route_sc_cache.append(functools.partial(
            pl.kernel,
            mesh=plsc.VectorSubcoreMesh(core_axis_name="c",
                                        subcore_axis_name="s"),
            out_type=[
                jax.ShapeDtypeStruct((NWORK, POOL, TPW), jnp.float32),
                jax.ShapeDtypeStruct((NWORK, TPW), jnp.float32),
            ],
            scratch_types=[
                pltpu.VMEM((2 * POOL, 2 * TPW), jnp.float32),
                pltpu.VMEM((POOL, TPW), jnp.float32),
                pltpu.VMEM((TPW,), jnp.float32),
            ],
        )(_route_sc_body))
    return _route_sc_cache[0]


def _pool_body(x_ref, w_ref, rz_ref, pat_ref, wi_ref, bi_ref, wo_ref,
               bo_ref, wq_ref, bq_ref, no_ref, qkv_ref):
    h = jax.nn.gelu(_bf(_dot(_bf(x_ref[...]), wi_ref[...]) + bi_ref[...]))
    w3 = w_ref[...]
    w = w3.transpose(0, 2, 1).reshape(w3.shape[0] * TPW, POOL)
    w = w * rz_ref[...]
    wp = _dot(_bf(w), pat_ref[...])
    fired = h * wp
    no = _dot(_bf(fired), wo_ref[...]) + bo_ref[...]
    no_ref[...] = no
    qkv_ref[...] = _bf(_dot(_bf(no), wq_ref[...]) + bq_ref[...])


def _attn_body(q_ref, k_ref, v_ref, wo_ref, bo_ref, x_ref, no_ref,
               ln1g_ref, ln1b_ref, ln2g_ref, ln2b_ref, out_ref):
    q_all = q_ref[...] * jnp.bfloat16(1.0 / (DH ** 0.5))
    k_all = k_ref[...]
    v_all = v_ref[...]
    ones = jnp.ones((S, DH), jnp.bfloat16)
    aos = []
    for h in range(HEADS):
        sl = slice(h * DH, (h + 1) * DH)
        s = _bf(_dot_t(q_all[:, sl], k_all[:, sl]))
        m = jnp.max(s, axis=-1, keepdims=True)
        p = jnp.exp(s - m)
        l = _dot(p, ones)[:, :1]
        aos.append(_bf(_dot(p, v_all[:, sl]) / l))

    attn_cat = jnp.concatenate(aos, axis=1)
    attn_out = _dot(attn_cat, wo_ref[...]) + bo_ref[...]
    t = x_ref[...] + attn_out
    mu = jnp.mean(t, axis=-1, keepdims=True)
    var = jnp.mean((t - mu) ** 2, axis=-1, keepdims=True)
    x1 = (t - mu) * lax.rsqrt(var + 1e-5) * ln1g_ref[...] + ln1b_ref[...]
    t2 = x1 + no_ref[...]
    mu2 = jnp.mean(t2, axis=-1, keepdims=True)
    var2 = jnp.mean((t2 - mu2) ** 2, axis=-1, keepdims=True)
    out_ref[...] = ((t2 - mu2) * lax.rsqrt(var2 + 1e-5) * ln2g_ref[...]
                    + ln2b_ref[...])


@jax.jit
def _run(x, W_router, b_router, patterns, mod_emb, W_ctx, b_ctx,
         W_in, b_in, W_out, b_out, W_qkv, b_qkv, W_o, b_o,
         ln1_g, ln1_b, ln2_g, ln2_b):
    x2 = x.reshape(S, D_MODEL)
    row = lambda v: v.reshape(1, -1)
    W_in_b = W_in.astype(jnp.bfloat16)
    patterns_b = patterns.astype(jnp.bfloat16)
    W_out_b = W_out.astype(jnp.bfloat16)
    W_qkv_b = W_qkv.astype(jnp.bfloat16)
    W_o_b = W_o.astype(jnp.bfloat16)

    full = lambda shape: pl.BlockSpec(shape, lambda i: (0, 0))
    seq = lambda w: pl.BlockSpec((BS, w), lambda i: (i, 0))

    cm = pl.pallas_call(
        _logits_body,
        grid=(S // BS,),
        in_specs=[
            seq(D_MODEL),
            full((D_MODEL, POOL)),
            full((1, POOL)),
            full((D_MODEL, MOD)),
            full((1, MOD)),
            full((POOL, MOD)),
        ],
        out_specs=pl.BlockSpec((2 * POOL, BS), lambda i: (0, i)),
        out_shape=jax.ShapeDtypeStruct((2 * POOL, S), jnp.float32),
        compiler_params=pltpu.CompilerParams(
            dimension_semantics=("arbitrary",)),
    )(x2, W_router, row(b_router), W_ctx, row(b_ctx), mod_emb)

    w3, rz = _get_route_sc()(cm)
    rz_col = rz.reshape(S, 1)

    neuron_output, qkv = pl.pallas_call(
        _pool_body,
        grid=(S // BS,),
        in_specs=[
            seq(D_MODEL),
            pl.BlockSpec((BS // TPW, POOL, TPW), lambda i: (i, 0, 0)),
            pl.BlockSpec((BS, 1), lambda i: (i, 0)),
            full((POOL, D_FF)),
            full((D_MODEL, D_FF)),
            full((1, D_FF)),
            full((D_FF, D_MODEL)),
            full((1, D_MODEL)),
            full((D_MODEL, 3 * D_MODEL)),
            full((1, 3 * D_MODEL)),
        ],
        out_specs=[seq(D_MODEL), seq(3 * D_MODEL)],
        out_shape=[jax.ShapeDtypeStruct((S, D_MODEL), jnp.float32),
                   jax.ShapeDtypeStruct((S, 3 * D_MODEL), jnp.bfloat16)],
        compiler_params=pltpu.CompilerParams(
            dimension_semantics=("arbitrary",)),
    )(x2, w3, rz_col, patterns_b, W_in_b, row(b_in), W_out_b, row(b_out),
      W_qkv_b, row(b_qkv))

    out = pl.pallas_call(
        _attn_body,
        grid=(S // QB,),
        in_specs=[
            pl.BlockSpec((QB, D_MODEL), lambda i: (i, 0)),
            pl.BlockSpec((S, D_MODEL), lambda i: (0, 1)),
            pl.BlockSpec((S, D_MODEL), lambda i: (0, 2)),
            full((D_MODEL, D_MODEL)),
            full((1, D_MODEL)),
            pl.BlockSpec((QB, D_MODEL), lambda i: (i, 0)),
            pl.BlockSpec((QB, D_MODEL), lambda i: (i, 0)),
            full((1, D_MODEL)),
            full((1, D_MODEL)),
            full((1, D_MODEL)),
            full((1, D_MODEL)),
        ],
        out_specs=pl.BlockSpec((QB, D_MODEL), lambda i: (i, 0)),
        out_shape=jax.ShapeDtypeStruct((S, D_MODEL), jnp.float32),
        compiler_params=pltpu.CompilerParams(
            dimension_semantics=("arbitrary",)),
    )(qkv, qkv, qkv, W_o_b, row(b_o), x2, neuron_output,
      row(ln1_g), row(ln1_b), row(ln2_g), row(ln2_b))

    return out.reshape(1, S, D_MODEL)


def kernel(x, W_router, b_router, patterns, mod_emb, W_ctx, b_ctx, W_in,
           b_in, W_out, b_out, W_qkv, b_qkv, W_o, b_o, ln1_g, ln1_b,
           ln2_g, ln2_b):
    return _run(x, W_router, b_router, patterns, mod_emb, W_ctx, b_ctx,
                W_in, b_in, W_out, b_out, W_qkv, b_qkv, W_o, b_o,
                ln1_g, ln1_b, ln2_g, ln2_b)


# speedup vs baseline: 1.0957x; 1.0957x over previous
import functools

import jax
import jax.numpy as jnp
from jax import lax
from jax.experimental import pallas as pl
from jax.experimental.pallas import tpu as pltpu
from jax.experimental.pallas import tpu_sc as plsc

D_MODEL = 1024
POOL = 64
TOPK = 8
D_FF = 2048
MOD = 64
HEADS = 16
DH = D_MODEL // HEADS
S = 2048

BS = 256
QB = 256
NEG = -1e30

NCORES = 2
NSUB = 16
NWORK = NCORES * NSUB
TPW = S // NWORK
LANES = 16


def _dot(a, b):
    return jax.lax.dot_general(a, b, (((1,), (0,)), ((), ())),
                               preferred_element_type=jnp.float32)


def _dot_t(a, b):
    return jax.lax.dot_general(a, b, (((1,), (1,)), ((), ())),
                               preferred_element_type=jnp.float32)


def _bf(t):
    return t.astype(jnp.bfloat16)


def _logits_body(x_ref, wr_ref, br_ref, wc_ref, bc_ref, me_ref, cm_ref):
    x = x_ref[...]
    scores = _dot(x, wr_ref[...]) + br_ref[...]
    ctx = _dot(x, wc_ref[...]) + bc_ref[...]
    modl = _dot_t(ctx, me_ref[...])
    cm_ref[...] = jnp.concatenate([scores.T, modl.T], axis=0)


def _route_sc_body(cm_hbm, w_hbm, rz_hbm, cm_v, out_v, rz_v):
    wid = lax.axis_index("s") * NCORES + lax.axis_index("c")
    slab = (wid // 2) * (2 * TPW)
    toff = (wid % 2) * TPW
    pltpu.sync_copy(cm_hbm.at[:, pl.ds(slab, 2 * TPW)], cm_v)

    for g in range(TPW // LANES):
        tok = pl.ds(toff + g * LANES, LANES)
        otok = pl.ds(g * LANES, LANES)
        neg = jnp.full((LANES,), NEG, jnp.float32)
        one = jnp.full((LANES,), 1, jnp.int32)
        zero = jnp.full((LANES,), 0, jnp.int32)

        def top_step(p, ms):
            ins = cm_v[p, tok]
            out = []
            for j in range(TOPK):
                hi = jnp.maximum(ms[j], ins)
                ins = jnp.minimum(ms[j], ins)
                out.append(hi)
            return tuple(out)

        ms = lax.fori_loop(0, POOL, top_step, tuple([neg] * TOPK),
                           unroll=False)
        m0 = ms[0]
        t8 = ms[TOPK - 1]
        ngt = jnp.zeros((LANES,), jnp.int32)
        for j in range(TOPK):
            ngt = ngt + jnp.where(ms[j] > t8, one, zero)
        quota = TOPK - ngt

        def sel_step(p, carry):
            z, c = carry
            v = cm_v[p, tok]
            eq = v == t8
            inc = jnp.logical_or(v > t8, jnp.logical_and(eq, c < quota))
            e = jnp.where(inc, jnp.exp(v - m0),
                          jnp.zeros((LANES,), jnp.float32))
            ml = cm_v[p + POOL, tok]
            sg = 1.0 / (1.0 + jnp.exp(-ml))
            out_v[p, otok] = e * sg
            return z + e, c + jnp.where(eq, one, zero)

        z, _ = lax.fori_loop(0, POOL, sel_step,
                             (jnp.zeros((LANES,), jnp.float32),
                              jnp.zeros((LANES,), jnp.int32)),
                             unroll=False)
        rz_v[otok] = 1.0 / z

    pltpu.sync_copy(out_v, w_hbm.at[wid])
    pltpu.sync_copy(rz_v, rz_hbm.at[wid])


_route_sc_cache = []


def _get_route_sc():
    if not _route_sc_cache:
        _route_sc_cache.append(functools.partial(
            pl.kernel,
            mesh=plsc.VectorSubcoreMesh(core_axis_name="c",
                                        subcore_axis_name="s"),
            out_type=[
                jax.ShapeDtypeStruct((NWORK, POOL, TPW), jnp.float32),
                jax.ShapeDtypeStruct((NWORK, TPW), jnp.float32),
            ],
            scratch_types=[
                pltpu.VMEM((2 * POOL, 2 * TPW), jnp.float32),
                pltpu.VMEM((POOL, TPW), jnp.float32),
                pltpu.VMEM((TPW,), jnp.float32),
            ],
        )(_route_sc_body))
    return _route_sc_cache[0]


def _pool_body(x_ref, w_ref, rz_ref, pat_ref, wi_ref, bi_ref, wo_ref,
               bo_ref, wq_ref, bq_ref, no_ref, qkv_ref):
    h = jax.nn.gelu(_bf(_dot(_bf(x_ref[...]), wi_ref[...]) + bi_ref[...]))
    w3 = w_ref[...]
    w = w3.transpose(0, 2, 1).reshape(w3.shape[0] * TPW, POOL)
    w = w * rz_ref[...]
    wp = _dot(_bf(w), pat_ref[...])
    fired = h * wp
    no = _dot(_bf(fired), wo_ref[...]) + bo_ref[...]
    no_ref[...] = no
    qkv_ref[...] = _bf(_dot(_bf(no), wq_ref[...]) + bq_ref[...])


def _attn_body(q_ref, k_ref, v_ref, wo_ref, bo_ref, x_ref, no_ref,
               ln1g_ref, ln1b_ref, ln2g_ref, ln2b_ref, out_ref):
    q_all = q_ref[...] * jnp.bfloat16(1.0 / (DH ** 0.5))
    k_all = k_ref[...]
    v_all = v_ref[...]
    aos = []
    for h in range(HEADS):
        sl = slice(h * DH, (h + 1) * DH)
        s = _bf(_dot_t(q_all[:, sl], k_all[:, sl]))
        m = jnp.max(s, axis=-1, keepdims=True)
        p = jnp.exp(s - m)
        l = jnp.sum(p, axis=-1, keepdims=True)
        aos.append(_bf(_dot(p, v_all[:, sl]) / l.astype(jnp.float32)))

    attn_cat = jnp.concatenate(aos, axis=1)
    attn_out = _dot(attn_cat, wo_ref[...]) + bo_ref[...]
    t = x_ref[...] + attn_out
    mu = jnp.mean(t, axis=-1, keepdims=True)
    var = jnp.mean((t - mu) ** 2, axis=-1, keepdims=True)
    x1 = (t - mu) * lax.rsqrt(var + 1e-5) * ln1g_ref[...] + ln1b_ref[...]
    t2 = x1 + no_ref[...]
    mu2 = jnp.mean(t2, axis=-1, keepdims=True)
    var2 = jnp.mean((t2 - mu2) ** 2, axis=-1, keepdims=True)
    out_ref[...] = ((t2 - mu2) * lax.rsqrt(var2 + 1e-5) * ln2g_ref[...]
                    + ln2b_ref[...])


@jax.jit
def _run(x, W_router, b_router, patterns, mod_emb, W_ctx, b_ctx,
         W_in, b_in, W_out, b_out, W_qkv, b_qkv, W_o, b_o,
         ln1_g, ln1_b, ln2_g, ln2_b):
    x2 = x.reshape(S, D_MODEL)
    row = lambda v: v.reshape(1, -1)
    W_in_b = W_in.astype(jnp.bfloat16)
    patterns_b = patterns.astype(jnp.bfloat16)
    W_out_b = W_out.astype(jnp.bfloat16)
    W_qkv_b = W_qkv.astype(jnp.bfloat16)
    W_o_b = W_o.astype(jnp.bfloat16)

    full = lambda shape: pl.BlockSpec(shape, lambda i: (0, 0))
    seq = lambda w: pl.BlockSpec((BS, w), lambda i: (i, 0))

    cm = pl.pallas_call(
        _logits_body,
        grid=(S // BS,),
        in_specs=[
            seq(D_MODEL),
            full((D_MODEL, POOL)),
            full((1, POOL)),
            full((D_MODEL, MOD)),
            full((1, MOD)),
            full((POOL, MOD)),
        ],
        out_specs=pl.BlockSpec((2 * POOL, BS), lambda i: (0, i)),
        out_shape=jax.ShapeDtypeStruct((2 * POOL, S), jnp.float32),
        compiler_params=pltpu.CompilerParams(
            dimension_semantics=("arbitrary",)),
    )(x2, W_router, row(b_router), W_ctx, row(b_ctx), mod_emb)

    w3, rz = _get_route_sc()(cm)
    rz_col = rz.reshape(S, 1)

    neuron_output, qkv = pl.pallas_call(
        _pool_body,
        grid=(S // BS,),
        in_specs=[
            seq(D_MODEL),
            pl.BlockSpec((BS // TPW, POOL, TPW), lambda i: (i, 0, 0)),
            pl.BlockSpec((BS, 1), lambda i: (i, 0)),
            full((POOL, D_FF)),
            full((D_MODEL, D_FF)),
            full((1, D_FF)),
            full((D_FF, D_MODEL)),
            full((1, D_MODEL)),
            full((D_MODEL, 3 * D_MODEL)),
            full((1, 3 * D_MODEL)),
        ],
        out_specs=[seq(D_MODEL), seq(3 * D_MODEL)],
        out_shape=[jax.ShapeDtypeStruct((S, D_MODEL), jnp.float32),
                   jax.ShapeDtypeStruct((S, 3 * D_MODEL), jnp.bfloat16)],
        compiler_params=pltpu.CompilerParams(
            dimension_semantics=("arbitrary",)),
    )(x2, w3, rz_col, patterns_b, W_in_b, row(b_in), W_out_b, row(b_out),
      W_qkv_b, row(b_qkv))

    out = pl.pallas_call(
        _attn_body,
        grid=(S // QB,),
        in_specs=[
            pl.BlockSpec((QB, D_MODEL), lambda i: (i, 0)),
            pl.BlockSpec((S, D_MODEL), lambda i: (0, 1)),
            pl.BlockSpec((S, D_MODEL), lambda i: (0, 2)),
            full((D_MODEL, D_MODEL)),
            full((1, D_MODEL)),
            pl.BlockSpec((QB, D_MODEL), lambda i: (i, 0)),
            pl.BlockSpec((QB, D_MODEL), lambda i: (i, 0)),
            full((1, D_MODEL)),
            full((1, D_MODEL)),
            full((1, D_MODEL)),
            full((1, D_MODEL)),
        ],
        out_specs=pl.BlockSpec((QB, D_MODEL), lambda i: (i, 0)),
        out_shape=jax.ShapeDtypeStruct((S, D_MODEL), jnp.float32),
        compiler_params=pltpu.CompilerParams(
            dimension_semantics=("arbitrary",)),
    )(qkv, qkv, qkv, W_o_b, row(b_o), x2, neuron_output,
      row(ln1_g), row(ln1_b), row(ln2_g), row(ln2_b))

    return out.reshape(1, S, D_MODEL)


def kernel(x, W_router, b_router, patterns, mod_emb, W_ctx, b_ctx, W_in,
           b_in, W_out, b_out, W_qkv, b_qkv, W_o, b_o, ln1_g, ln1_b,
           ln2_g, ln2_b):
    return _run(x, W_router, b_router, patterns, mod_emb, W_ctx, b_ctx,
                W_in, b_in, W_out, b_out, W_qkv, b_qkv, W_o, b_o,
                ln1_g, ln1_b, ln2_g, ln2_b)


# speedup vs baseline: 1.1114x; 1.0144x over previous
import functools

import jax
import jax.numpy as jnp
from jax import lax
from jax.experimental import pallas as pl
from jax.experimental.pallas import tpu as pltpu
from jax.experimental.pallas import tpu_sc as plsc

D_MODEL = 1024
POOL = 64
TOPK = 8
D_FF = 2048
MOD = 64
HEADS = 16
DH = D_MODEL // HEADS
S = 2048

BS = 256
QB = 256
NEG = -1e30

NCORES = 2
NSUB = 16
NWORK = NCORES * NSUB
TPW = S // NWORK
LANES = 16


def _dot(a, b):
    return jax.lax.dot_general(a, b, (((1,), (0,)), ((), ())),
                               preferred_element_type=jnp.float32)


def _dot_t(a, b):
    return jax.lax.dot_general(a, b, (((1,), (1,)), ((), ())),
                               preferred_element_type=jnp.float32)


def _bf(t):
    return t.astype(jnp.bfloat16)


def _logits_body(x_ref, wr_ref, br_ref, wc_ref, bc_ref, me_ref, cm_ref):
    x = x_ref[...]
    scores = _dot(x, wr_ref[...]) + br_ref[...]
    ctx = _dot(x, wc_ref[...]) + bc_ref[...]
    modl = _dot_t(ctx, me_ref[...])
    cm_ref[...] = jnp.concatenate([scores.T, modl.T], axis=0)


def _route_sc_body(cm_hbm, w_hbm, cm_v, out_v):
    wid = lax.axis_index("s") * NCORES + lax.axis_index("c")
    slab = (wid // 2) * (2 * TPW)
    toff = (wid % 2) * TPW
    pltpu.sync_copy(cm_hbm.at[:, pl.ds(slab, 2 * TPW)], cm_v)

    for g in range(TPW // LANES):
        tok = pl.ds(toff + g * LANES, LANES)
        otok = pl.ds(g * LANES, LANES)
        neg = jnp.full((LANES,), NEG, jnp.float32)
        one = jnp.full((LANES,), 1, jnp.int32)
        zero = jnp.full((LANES,), 0, jnp.int32)

        def top_step(p, ms):
            ins = cm_v[p, tok]
            out = []
            for j in range(TOPK):
                hi = jnp.maximum(ms[j], ins)
                ins = jnp.minimum(ms[j], ins)
                out.append(hi)
            return tuple(out)

        ms = lax.fori_loop(0, POOL, top_step, tuple([neg] * TOPK),
                           unroll=False)
        m0 = ms[0]
        t8 = ms[TOPK - 1]
        ngt = jnp.zeros((LANES,), jnp.int32)
        for j in range(TOPK):
            ngt = ngt + jnp.where(ms[j] > t8, one, zero)
        quota = TOPK - ngt

        def sel_step(p, carry):
            z, c = carry
            v = cm_v[p, tok]
            eq = v == t8
            inc = jnp.logical_or(v > t8, jnp.logical_and(eq, c < quota))
            e = jnp.where(inc, jnp.exp(v - m0),
                          jnp.zeros((LANES,), jnp.float32))
            ml = cm_v[p + POOL, tok]
            sg = 1.0 / (1.0 + jnp.exp(-ml))
            out_v[p, otok] = e * sg
            return z + e, c + jnp.where(eq, one, zero)

        z, _ = lax.fori_loop(0, POOL, sel_step,
                             (jnp.zeros((LANES,), jnp.float32),
                              jnp.zeros((LANES,), jnp.int32)),
                             unroll=False)
        rz = 1.0 / z

        def norm_step(p, carry):
            out_v[p, otok] = out_v[p, otok] * rz
            return carry

        lax.fori_loop(0, POOL, norm_step, 0, unroll=False)

    pltpu.sync_copy(out_v, w_hbm.at[wid])


_route_sc_cache = []


def _get_route_sc():
    if not _route_sc_cache:
        _route_sc_cache.append(functools.partial(
            pl.kernel,
            mesh=plsc.VectorSubcoreMesh(core_axis_name="c",
                                        subcore_axis_name="s"),
            out_type=jax.ShapeDtypeStruct((NWORK, POOL, TPW), jnp.float32),
            scratch_types=[
                pltpu.VMEM((2 * POOL, 2 * TPW), jnp.float32),
                pltpu.VMEM((POOL, TPW), jnp.float32),
            ],
        )(_route_sc_body))
    return _route_sc_cache[0]


def _pool_body(x_ref, w_ref, pat_ref, wi_ref, bi_ref, wo_ref,
               bo_ref, wq_ref, bq_ref, no_ref, qkv_ref):
    h = jax.nn.gelu(_bf(_dot(_bf(x_ref[...]), wi_ref[...]) + bi_ref[...]))
    w3 = w_ref[...]
    w = w3.transpose(0, 2, 1).reshape(w3.shape[0] * TPW, POOL)
    wp = _dot(_bf(w), pat_ref[...])
    fired = h * wp
    no = _dot(_bf(fired), wo_ref[...]) + bo_ref[...]
    no_ref[...] = no
    qkv_ref[...] = _bf(_dot(_bf(no), wq_ref[...]) + bq_ref[...])


def _attn_body(q_ref, k_ref, v_ref, wo_ref, bo_ref, x_ref, no_ref,
               ln1g_ref, ln1b_ref, ln2g_ref, ln2b_ref, out_ref):
    q_all = q_ref[...] * jnp.bfloat16(1.0 / (DH ** 0.5))
    k_all = k_ref[...]
    v_all = v_ref[...]
    aos = []
    for h in range(HEADS):
        sl = slice(h * DH, (h + 1) * DH)
        s = _bf(_dot_t(q_all[:, sl], k_all[:, sl]))
        m = jnp.max(s, axis=-1, keepdims=True)
        p = jnp.exp(s - m)
        l = jnp.sum(p, axis=-1, keepdims=True)
        aos.append(_bf(_dot(p, v_all[:, sl]) / l.astype(jnp.float32)))

    attn_cat = jnp.concatenate(aos, axis=1)
    attn_out = _dot(attn_cat, wo_ref[...]) + bo_ref[...]
    t = x_ref[...] + attn_out
    mu = jnp.mean(t, axis=-1, keepdims=True)
    var = jnp.mean((t - mu) ** 2, axis=-1, keepdims=True)
    x1 = (t - mu) * lax.rsqrt(var + 1e-5) * ln1g_ref[...] + ln1b_ref[...]
    t2 = x1 + no_ref[...]
    mu2 = jnp.mean(t2, axis=-1, keepdims=True)
    var2 = jnp.mean((t2 - mu2) ** 2, axis=-1, keepdims=True)
    out_ref[...] = ((t2 - mu2) * lax.rsqrt(var2 + 1e-5) * ln2g_ref[...]
                    + ln2b_ref[...])


@jax.jit
def _run(x, W_router, b_router, patterns, mod_emb, W_ctx, b_ctx,
         W_in, b_in, W_out, b_out, W_qkv, b_qkv, W_o, b_o,
         ln1_g, ln1_b, ln2_g, ln2_b):
    x2 = x.reshape(S, D_MODEL)
    row = lambda v: v.reshape(1, -1)
    W_in_b = W_in.astype(jnp.bfloat16)
    patterns_b = patterns.astype(jnp.bfloat16)
    W_out_b = W_out.astype(jnp.bfloat16)
    W_qkv_b = W_qkv.astype(jnp.bfloat16)
    W_o_b = W_o.astype(jnp.bfloat16)

    full = lambda shape: pl.BlockSpec(shape, lambda i: (0, 0))
    seq = lambda w: pl.BlockSpec((BS, w), lambda i: (i, 0))

    cm = pl.pallas_call(
        _logits_body,
        grid=(S // BS,),
        in_specs=[
            seq(D_MODEL),
            full((D_MODEL, POOL)),
            full((1, POOL)),
            full((D_MODEL, MOD)),
            full((1, MOD)),
            full((POOL, MOD)),
        ],
        out_specs=pl.BlockSpec((2 * POOL, BS), lambda i: (0, i)),
        out_shape=jax.ShapeDtypeStruct((2 * POOL, S), jnp.float32),
        compiler_params=pltpu.CompilerParams(
            dimension_semantics=("arbitrary",)),
    )(x2, W_router, row(b_router), W_ctx, row(b_ctx), mod_emb)

    w3 = _get_route_sc()(cm)

    neuron_output, qkv = pl.pallas_call(
        _pool_body,
        grid=(S // BS,),
        in_specs=[
            seq(D_MODEL),
            pl.BlockSpec((BS // TPW, POOL, TPW), lambda i: (i, 0, 0)),
            full((POOL, D_FF)),
            full((D_MODEL, D_FF)),
            full((1, D_FF)),
            full((D_FF, D_MODEL)),
            full((1, D_MODEL)),
            full((D_MODEL, 3 * D_MODEL)),
            full((1, 3 * D_MODEL)),
        ],
        out_specs=[seq(D_MODEL), seq(3 * D_MODEL)],
        out_shape=[jax.ShapeDtypeStruct((S, D_MODEL), jnp.float32),
                   jax.ShapeDtypeStruct((S, 3 * D_MODEL), jnp.bfloat16)],
        compiler_params=pltpu.CompilerParams(
            dimension_semantics=("arbitrary",)),
    )(x2, w3, patterns_b, W_in_b, row(b_in), W_out_b, row(b_out),
      W_qkv_b, row(b_qkv))

    out = pl.pallas_call(
        _attn_body,
        grid=(S // QB,),
        in_specs=[
            pl.BlockSpec((QB, D_MODEL), lambda i: (i, 0)),
            pl.BlockSpec((S, D_MODEL), lambda i: (0, 1)),
            pl.BlockSpec((S, D_MODEL), lambda i: (0, 2)),
            full((D_MODEL, D_MODEL)),
            full((1, D_MODEL)),
            pl.BlockSpec((QB, D_MODEL), lambda i: (i, 0)),
            pl.BlockSpec((QB, D_MODEL), lambda i: (i, 0)),
            full((1, D_MODEL)),
            full((1, D_MODEL)),
            full((1, D_MODEL)),
            full((1, D_MODEL)),
        ],
        out_specs=pl.BlockSpec((QB, D_MODEL), lambda i: (i, 0)),
        out_shape=jax.ShapeDtypeStruct((S, D_MODEL), jnp.float32),
        compiler_params=pltpu.CompilerParams(
            dimension_semantics=("arbitrary",)),
    )(qkv, qkv, qkv, W_o_b, row(b_o), x2, neuron_output,
      row(ln1_g), row(ln1_b), row(ln2_g), row(ln2_b))

    return out.reshape(1, S, D_MODEL)


def kernel(x, W_router, b_router, patterns, mod_emb, W_ctx, b_ctx, W_in,
           b_in, W_out, b_out, W_qkv, b_qkv, W_o, b_o, ln1_g, ln1_b,
           ln2_g, ln2_b):
    return _run(x, W_router, b_router, patterns, mod_emb, W_ctx, b_ctx,
                W_in, b_in, W_out, b_out, W_qkv, b_qkv, W_o, b_o,
                ln1_g, ln1_b, ln2_g, ln2_b)


# speedup vs baseline: 1.1164x; 1.0045x over previous
import functools

import jax
import jax.numpy as jnp
from jax import lax
from jax.experimental import pallas as pl
from jax.experimental.pallas import tpu as pltpu
from jax.experimental.pallas import tpu_sc as plsc

D_MODEL = 1024
POOL = 64
TOPK = 8
D_FF = 2048
MOD = 64
HEADS = 16
DH = D_MODEL // HEADS
S = 2048

BS = 256
PB = 512
QB = 256
NEG = -1e30

NCORES = 2
NSUB = 16
NWORK = NCORES * NSUB
TPW = S // NWORK
LANES = 16


def _dot(a, b):
    return jax.lax.dot_general(a, b, (((1,), (0,)), ((), ())),
                               preferred_element_type=jnp.float32)


def _dot_t(a, b):
    return jax.lax.dot_general(a, b, (((1,), (1,)), ((), ())),
                               preferred_element_type=jnp.float32)


def _bf(t):
    return t.astype(jnp.bfloat16)


def _logits_body(x_ref, wr_ref, br_ref, wc_ref, bc_ref, me_ref, cm_ref):
    x = x_ref[...]
    scores = _dot(x, wr_ref[...]) + br_ref[...]
    ctx = _dot(x, wc_ref[...]) + bc_ref[...]
    modl = _dot_t(ctx, me_ref[...])
    cm_ref[...] = jnp.concatenate([scores.T, modl.T], axis=0)


def _route_sc_body(cm_hbm, w_hbm, cm_v, out_v):
    wid = lax.axis_index("s") * NCORES + lax.axis_index("c")
    slab = (wid // 2) * (2 * TPW)
    toff = (wid % 2) * TPW
    pltpu.sync_copy(cm_hbm.at[:, pl.ds(slab, 2 * TPW)], cm_v)

    for g in range(TPW // LANES):
        tok = pl.ds(toff + g * LANES, LANES)
        otok = pl.ds(g * LANES, LANES)
        neg = jnp.full((LANES,), NEG, jnp.float32)
        one = jnp.full((LANES,), 1, jnp.int32)
        zero = jnp.full((LANES,), 0, jnp.int32)

        def top_step(p, ms):
            ins = cm_v[p, tok]
            out = []
            for j in range(TOPK):
                hi = jnp.maximum(ms[j], ins)
                ins = jnp.minimum(ms[j], ins)
                out.append(hi)
            return tuple(out)

        ms = lax.fori_loop(0, POOL, top_step, tuple([neg] * TOPK),
                           unroll=False)
        m0 = ms[0]
        t8 = ms[TOPK - 1]
        ngt = jnp.zeros((LANES,), jnp.int32)
        for j in range(TOPK):
            ngt = ngt + jnp.where(ms[j] > t8, one, zero)
        quota = TOPK - ngt

        def sel_step(p, carry):
            z, c = carry
            v = cm_v[p, tok]
            eq = v == t8
            inc = jnp.logical_or(v > t8, jnp.logical_and(eq, c < quota))
            e = jnp.where(inc, jnp.exp(v - m0),
                          jnp.zeros((LANES,), jnp.float32))
            ml = cm_v[p + POOL, tok]
            sg = 1.0 / (1.0 + jnp.exp(-ml))
            out_v[p, otok] = e * sg
            return z + e, c + jnp.where(eq, one, zero)

        z, _ = lax.fori_loop(0, POOL, sel_step,
                             (jnp.zeros((LANES,), jnp.float32),
                              jnp.zeros((LANES,), jnp.int32)),
                             unroll=False)
        rz = 1.0 / z

        def norm_step(p, carry):
            out_v[p, otok] = out_v[p, otok] * rz
            return carry

        lax.fori_loop(0, POOL, norm_step, 0, unroll=False)

    pltpu.sync_copy(out_v, w_hbm.at[wid])


_route_sc_cache = []


def _get_route_sc():
    if not _route_sc_cache:
        _route_sc_cache.append(functools.partial(
            pl.kernel,
            mesh=plsc.VectorSubcoreMesh(core_axis_name="c",
                                        subcore_axis_name="s"),
            out_type=jax.ShapeDtypeStruct((NWORK, POOL, TPW), jnp.float32),
            scratch_types=[
                pltpu.VMEM((2 * POOL, 2 * TPW), jnp.float32),
                pltpu.VMEM((POOL, TPW), jnp.float32),
            ],
        )(_route_sc_body))
    return _route_sc_cache[0]


def _pool_body(x_ref, w_ref, pat_ref, wi_ref, bi_ref, wo_ref,
               bo_ref, wq_ref, bq_ref, no_ref, qkv_ref):
    h = jax.nn.gelu(_bf(_dot(_bf(x_ref[...]), wi_ref[...]) + bi_ref[...]))
    w3 = w_ref[...]
    w = w3.transpose(0, 2, 1).reshape(w3.shape[0] * TPW, POOL)
    wp = _dot(_bf(w), pat_ref[...])
    fired = h * wp
    no = _dot(_bf(fired), wo_ref[...]) + bo_ref[...]
    no_ref[...] = no
    qkv_ref[...] = _bf(_dot(_bf(no), wq_ref[...]) + bq_ref[...])


def _attn_body(q_ref, k_ref, v_ref, wo_ref, bo_ref, x_ref, no_ref,
               ln1g_ref, ln1b_ref, ln2g_ref, ln2b_ref, out_ref):
    q_all = q_ref[...] * jnp.bfloat16(1.0 / (DH ** 0.5))
    k_all = k_ref[...]
    v_all = v_ref[...]
    aos = []
    for h in range(HEADS):
        sl = slice(h * DH, (h + 1) * DH)
        s = _bf(_dot_t(q_all[:, sl], k_all[:, sl]))
        m = jnp.max(s, axis=-1, keepdims=True)
        p = jnp.exp(s - m)
        l = jnp.sum(p, axis=-1, keepdims=True)
        aos.append(_bf(_dot(p, v_all[:, sl]) / l.astype(jnp.float32)))

    attn_cat = jnp.concatenate(aos, axis=1)
    attn_out = _dot(attn_cat, wo_ref[...]) + bo_ref[...]
    t = x_ref[...] + attn_out
    mu = jnp.mean(t, axis=-1, keepdims=True)
    var = jnp.mean((t - mu) ** 2, axis=-1, keepdims=True)
    x1 = (t - mu) * lax.rsqrt(var + 1e-5) * ln1g_ref[...] + ln1b_ref[...]
    t2 = x1 + no_ref[...]
    mu2 = jnp.mean(t2, axis=-1, keepdims=True)
    var2 = jnp.mean((t2 - mu2) ** 2, axis=-1, keepdims=True)
    out_ref[...] = ((t2 - mu2) * lax.rsqrt(var2 + 1e-5) * ln2g_ref[...]
                    + ln2b_ref[...])


@jax.jit
def _run(x, W_router, b_router, patterns, mod_emb, W_ctx, b_ctx,
         W_in, b_in, W_out, b_out, W_qkv, b_qkv, W_o, b_o,
         ln1_g, ln1_b, ln2_g, ln2_b):
    x2 = x.reshape(S, D_MODEL)
    row = lambda v: v.reshape(1, -1)
    W_in_b = W_in.astype(jnp.bfloat16)
    patterns_b = patterns.astype(jnp.bfloat16)
    W_out_b = W_out.astype(jnp.bfloat16)
    W_qkv_b = W_qkv.astype(jnp.bfloat16)
    W_o_b = W_o.astype(jnp.bfloat16)

    full = lambda shape: pl.BlockSpec(shape, lambda i: (0, 0))
    seq = lambda w: pl.BlockSpec((BS, w), lambda i: (i, 0))

    cm = pl.pallas_call(
        _logits_body,
        grid=(S // BS,),
        in_specs=[
            seq(D_MODEL),
            full((D_MODEL, POOL)),
            full((1, POOL)),
            full((D_MODEL, MOD)),
            full((1, MOD)),
            full((POOL, MOD)),
        ],
        out_specs=pl.BlockSpec((2 * POOL, BS), lambda i: (0, i)),
        out_shape=jax.ShapeDtypeStruct((2 * POOL, S), jnp.float32),
        compiler_params=pltpu.CompilerParams(
            dimension_semantics=("arbitrary",)),
    )(x2, W_router, row(b_router), W_ctx, row(b_ctx), mod_emb)

    w3 = _get_route_sc()(cm)

    pseq = lambda w: pl.BlockSpec((PB, w), lambda i: (i, 0))
    neuron_output, qkv = pl.pallas_call(
        _pool_body,
        grid=(S // PB,),
        in_specs=[
            pseq(D_MODEL),
            pl.BlockSpec((PB // TPW, POOL, TPW), lambda i: (i, 0, 0)),
            full((POOL, D_FF)),
            full((D_MODEL, D_FF)),
            full((1, D_FF)),
            full((D_FF, D_MODEL)),
            full((1, D_MODEL)),
            full((D_MODEL, 3 * D_MODEL)),
            full((1, 3 * D_MODEL)),
        ],
        out_specs=[pseq(D_MODEL), pseq(3 * D_MODEL)],
        out_shape=[jax.ShapeDtypeStruct((S, D_MODEL), jnp.float32),
                   jax.ShapeDtypeStruct((S, 3 * D_MODEL), jnp.bfloat16)],
        compiler_params=pltpu.CompilerParams(
            dimension_semantics=("arbitrary",)),
    )(x2, w3, patterns_b, W_in_b, row(b_in), W_out_b, row(b_out),
      W_qkv_b, row(b_qkv))

    out = pl.pallas_call(
        _attn_body,
        grid=(S // QB,),
        in_specs=[
            pl.BlockSpec((QB, D_MODEL), lambda i: (i, 0)),
            pl.BlockSpec((S, D_MODEL), lambda i: (0, 1)),
            pl.BlockSpec((S, D_MODEL), lambda i: (0, 2)),
            full((D_MODEL, D_MODEL)),
            full((1, D_MODEL)),
            pl.BlockSpec((QB, D_MODEL), lambda i: (i, 0)),
            pl.BlockSpec((QB, D_MODEL), lambda i: (i, 0)),
            full((1, D_MODEL)),
            full((1, D_MODEL)),
            full((1, D_MODEL)),
            full((1, D_MODEL)),
        ],
        out_specs=pl.BlockSpec((QB, D_MODEL), lambda i: (i, 0)),
        out_shape=jax.ShapeDtypeStruct((S, D_MODEL), jnp.float32),
        compiler_params=pltpu.CompilerParams(
            dimension_semantics=("arbitrary",)),
    )(qkv, qkv, qkv, W_o_b, row(b_o), x2, neuron_output,
      row(ln1_g), row(ln1_b), row(ln2_g), row(ln2_b))

    return out.reshape(1, S, D_MODEL)


def kernel(x, W_router, b_router, patterns, mod_emb, W_ctx, b_ctx, W_in,
           b_in, W_out, b_out, W_qkv, b_qkv, W_o, b_o, ln1_g, ln1_b,
           ln2_g, ln2_b):
    return _run(x, W_router, b_router, patterns, mod_emb, W_ctx, b_ctx,
                W_in, b_in, W_out, b_out, W_qkv, b_qkv, W_o, b_o,
                ln1_g, ln1_b, ln2_g, ln2_b)
